# baseline (device time: 44946 ns/iter reference)
import jax
import jax.numpy as jnp
from jax import lax
from jax.experimental import pallas as pl
from jax.experimental.pallas import tpu as pltpu

N_DEV = 4
HQ_GLOBAL = 16
DH = 64
HG = HQ_GLOBAL // N_DEV
GD = HG * DH
BLOCK = 64
NBLK = 4


def kernel(x, Wq, K_ext, V_ext, Wo):
    B_loc, Sq, Dm = x.shape
    Skv = K_ext.shape[1]
    R = B_loc * Sq
    assert Sq == Skv == NBLK * BLOCK

    def body(x_ref, wq_ref, k_hbm, v_hbm, wo_ref, out_ref,
             x16s, wq16, wo16, qbuf, obuf, k_vmem, v_vmem,
             send_sems, recv_sems, k_sems, v_sems):
        my_pos = lax.axis_index("i")

        kv_copies = []
        for e in range(N_DEV):
            g = (my_pos - e) % N_DEV
            rows = pl.ds(my_pos * B_loc, B_loc)
            slot = []
            for hh in range(HG):
                head = g * HG + hh
                ck = pltpu.make_async_copy(
                    k_hbm.at[rows, :, head, :],
                    k_vmem.at[e * HG + hh], k_sems.at[e])
                cv = pltpu.make_async_copy(
                    v_hbm.at[rows, :, head, :],
                    v_vmem.at[e * HG + hh], v_sems.at[e])
                ck.start()
                cv.start()
                slot.append((ck, cv))
            kv_copies.append(slot)

        x16s[...] = x_ref[...].reshape(R, Dm).astype(jnp.bfloat16)
        wq16[...] = wq_ref[...].astype(jnp.bfloat16)
        wo16[...] = wo_ref[...].astype(jnp.bfloat16)

        barrier = pltpu.get_barrier_semaphore()
        for d in (1, 2, 3):
            pl.semaphore_signal(
                barrier, inc=1,
                device_id=((my_pos + d) % N_DEV,),
                device_id_type=pl.DeviceIdType.MESH,
            )
        pl.semaphore_wait(barrier, 3)

        sends = []
        for w, src in ((0, wq16), (1, wo16)):
            for d in (1, 2, 3):
                buf = qbuf if w == 0 else obuf
                rdma = pltpu.make_async_remote_copy(
                    src_ref=src,
                    dst_ref=buf.at[d - 1],
                    send_sem=send_sems.at[w * 3 + d - 1],
                    recv_sem=recv_sems.at[w * 3 + d - 1],
                    device_id=((my_pos + d) % N_DEV,),
                    device_id_type=pl.DeviceIdType.MESH,
                )
                rdma.start()
                sends.append(rdma)

        def recv_wait(w, e):
            buf = qbuf if w == 0 else obuf
            pltpu.make_async_remote_copy(
                src_ref=wq16 if w == 0 else wo16,
                dst_ref=buf.at[e - 1],
                send_sem=send_sems.at[0],
                recv_sem=recv_sems.at[w * 3 + e - 1],
                device_id=(my_pos,),
                device_id_type=pl.DeviceIdType.MESH,
            ).wait_recv()

        xv = x16s[...]

        def qk_attn(wq_v, e):
            q = lax.dot_general(
                xv, wq_v, (((1,), (0,)), ((), ())),
                preferred_element_type=jnp.float32)
            for ck, cv in kv_copies[e]:
                ck.wait()
                cv.wait()
            ctxs = []
            for hh in range(HG):
                sl = slice(hh * DH, (hh + 1) * DH)
                qb = q[:, sl].reshape(B_loc * NBLK, BLOCK, DH)
                kb = k_vmem[e * HG + hh].reshape(B_loc * NBLK, BLOCK, DH)
                vb = v_vmem[e * HG + hh].reshape(B_loc * NBLK, BLOCK, DH)
                s = lax.dot_general(
                    qb, kb, (((2,), (2,)), ((0,), (0,))),
                    preferred_element_type=jnp.float32) * 0.125
                m = jnp.max(s, axis=2, keepdims=True)
                p = jnp.exp(s - m)
                p = p / jnp.sum(p, axis=2, keepdims=True)
                ctx = lax.dot_general(
                    p, vb, (((2,), (1,)), ((0,), (0,))),
                    preferred_element_type=jnp.float32)
                ctxs.append(ctx.reshape(R, DH))
            return jnp.concatenate(ctxs, axis=1).astype(jnp.bfloat16)

        def out_proj(ctx16, wo_v, first):
            contrib = lax.dot_general(
                ctx16, wo_v, (((1,), (0,)), ((), ())),
                preferred_element_type=jnp.float32)
            if first:
                out_ref[...] = contrib
            else:
                out_ref[...] = out_ref[...] + contrib

        out_proj(qk_attn(wq16[...], 0), wo16[...], first=True)

        order = (1, 3, 2)
        ctx_by_e = {}
        for e in order:
            recv_wait(0, e)
            ctx_by_e[e] = qk_attn(qbuf[e - 1], e)

        for e in order:
            recv_wait(1, e)
            out_proj(ctx_by_e[e], obuf[e - 1], first=False)

        for rdma in sends:
            rdma.wait_send()

    out = pl.pallas_call(
        body,
        out_shape=jax.ShapeDtypeStruct((R, Dm), jnp.float32),
        in_specs=[
            pl.BlockSpec(memory_space=pltpu.VMEM),
            pl.BlockSpec(memory_space=pltpu.VMEM),
            pl.BlockSpec(memory_space=pltpu.MemorySpace.HBM),
            pl.BlockSpec(memory_space=pltpu.MemorySpace.HBM),
            pl.BlockSpec(memory_space=pltpu.VMEM),
        ],
        out_specs=pl.BlockSpec(memory_space=pltpu.VMEM),
        scratch_shapes=[
            pltpu.VMEM((R, Dm), jnp.bfloat16),
            pltpu.VMEM((Dm, GD), jnp.bfloat16),
            pltpu.VMEM((GD, Dm), jnp.bfloat16),
            pltpu.VMEM((3, Dm, GD), jnp.bfloat16),
            pltpu.VMEM((3, GD, Dm), jnp.bfloat16),
            pltpu.VMEM((N_DEV * HG, B_loc, Skv, DH), jnp.float32),
            pltpu.VMEM((N_DEV * HG, B_loc, Skv, DH), jnp.float32),
            pltpu.SemaphoreType.DMA((6,)),
            pltpu.SemaphoreType.DMA((6,)),
            pltpu.SemaphoreType.DMA((N_DEV,)),
            pltpu.SemaphoreType.DMA((N_DEV,)),
        ],
        compiler_params=pltpu.CompilerParams(collective_id=0),
    )(x, Wq, K_ext, V_ext, Wo)
    return out.reshape(B_loc, Sq, Dm)


# device time: 21551 ns/iter; 2.0856x vs baseline; 2.0856x over previous
import jax
import jax.numpy as jnp
from jax import lax
from jax.experimental import pallas as pl
from jax.experimental.pallas import tpu as pltpu

N_DEV = 4
HQ_GLOBAL = 16
DH = 64
HG = HQ_GLOBAL // N_DEV
GD = HG * DH
BLOCK = 64
NBLK = 4


def kernel(x, Wq, K_ext, V_ext, Wo):
    B_loc, Sq, Dm = x.shape
    Skv = K_ext.shape[1]
    R = B_loc * Sq
    assert Sq == Skv == NBLK * BLOCK
    my = lax.axis_index("i")

    K_g = (lax.dynamic_slice_in_dim(K_ext, my * B_loc, B_loc, axis=0)
           .reshape(B_loc, Skv, N_DEV, GD).transpose(2, 0, 1, 3)
           .astype(jnp.bfloat16))
    V_g = (lax.dynamic_slice_in_dim(V_ext, my * B_loc, B_loc, axis=0)
           .reshape(B_loc, Skv, N_DEV, GD).transpose(2, 0, 1, 3)
           .astype(jnp.bfloat16))

    def body(x_ref, wq_ref, k_ref, v_ref, wo_ref, out_ref,
             x16s, wq16, wo16, qbuf, obuf, send_sems, recv_sems):
        my_pos = lax.axis_index("i")

        x16s[...] = x_ref[...].reshape(R, Dm).astype(jnp.bfloat16)
        wq16[...] = wq_ref[...].astype(jnp.bfloat16)
        wo16[...] = wo_ref[...].astype(jnp.bfloat16)

        barrier = pltpu.get_barrier_semaphore()
        for d in (1, 2, 3):
            pl.semaphore_signal(
                barrier, inc=1,
                device_id=((my_pos + d) % N_DEV,),
                device_id_type=pl.DeviceIdType.MESH,
            )
        pl.semaphore_wait(barrier, 3)

        sends = []
        for w, src in ((0, wq16), (1, wo16)):
            for d in (1, 2, 3):
                buf = qbuf if w == 0 else obuf
                rdma = pltpu.make_async_remote_copy(
                    src_ref=src,
                    dst_ref=buf.at[d - 1],
                    send_sem=send_sems.at[w * 3 + d - 1],
                    recv_sem=recv_sems.at[w * 3 + d - 1],
                    device_id=((my_pos + d) % N_DEV,),
                    device_id_type=pl.DeviceIdType.MESH,
                )
                rdma.start()
                sends.append(rdma)

        def recv_wait(w, e):
            buf = qbuf if w == 0 else obuf
            pltpu.make_async_remote_copy(
                src_ref=wq16 if w == 0 else wo16,
                dst_ref=buf.at[e - 1],
                send_sem=send_sems.at[0],
                recv_sem=recv_sems.at[w * 3 + e - 1],
                device_id=(my_pos,),
                device_id_type=pl.DeviceIdType.MESH,
            ).wait_recv()

        xv = x16s[...]

        def qk_attn(wq_v, group):
            q16 = lax.dot_general(
                xv, wq_v, (((1,), (0,)), ((), ())),
                preferred_element_type=jnp.float32,
                ).astype(jnp.bfloat16)
            kg = k_ref[group]
            vg = v_ref[group]
            ctxs = []
            for hh in range(HG):
                sl = slice(hh * DH, (hh + 1) * DH)
                qb = q16[:, sl].reshape(B_loc * NBLK, BLOCK, DH)
                kb = kg[:, :, sl].reshape(B_loc * NBLK, BLOCK, DH)
                vb = vg[:, :, sl].reshape(B_loc * NBLK, BLOCK, DH)
                s = lax.dot_general(
                    qb, kb, (((2,), (2,)), ((0,), (0,))),
                    preferred_element_type=jnp.float32) * 0.125
                m = jnp.max(s, axis=2, keepdims=True)
                p = jnp.exp(s - m)
                p = (p / jnp.sum(p, axis=2, keepdims=True)).astype(jnp.bfloat16)
                ctx = lax.dot_general(
                    p, vb, (((2,), (1,)), ((0,), (0,))),
                    preferred_element_type=jnp.float32)
                ctxs.append(ctx.reshape(R, DH))
            return jnp.concatenate(ctxs, axis=1).astype(jnp.bfloat16)

        def out_proj(ctx16, wo_v, first):
            contrib = lax.dot_general(
                ctx16, wo_v, (((1,), (0,)), ((), ())),
                preferred_element_type=jnp.float32)
            if first:
                out_ref[...] = contrib
            else:
                out_ref[...] = out_ref[...] + contrib

        out_proj(qk_attn(wq16[...], my_pos), wo16[...], first=True)

        order = (1, 3, 2)
        ctx_by_e = {}
        for e in order:
            recv_wait(0, e)
            ctx_by_e[e] = qk_attn(qbuf[e - 1], (my_pos - e) % N_DEV)

        for e in order:
            recv_wait(1, e)
            out_proj(ctx_by_e[e], obuf[e - 1], first=False)

        for rdma in sends:
            rdma.wait_send()

    out = pl.pallas_call(
        body,
        out_shape=jax.ShapeDtypeStruct((R, Dm), jnp.float32),
        in_specs=[pl.BlockSpec(memory_space=pltpu.VMEM)] * 5,
        out_specs=pl.BlockSpec(memory_space=pltpu.VMEM),
        scratch_shapes=[
            pltpu.VMEM((R, Dm), jnp.bfloat16),
            pltpu.VMEM((Dm, GD), jnp.bfloat16),
            pltpu.VMEM((GD, Dm), jnp.bfloat16),
            pltpu.VMEM((3, Dm, GD), jnp.bfloat16),
            pltpu.VMEM((3, GD, Dm), jnp.bfloat16),
            pltpu.SemaphoreType.DMA((6,)),
            pltpu.SemaphoreType.DMA((6,)),
        ],
        compiler_params=pltpu.CompilerParams(collective_id=0),
    )(x, Wq, K_g, V_g, Wo)
    return out.reshape(B_loc, Sq, Dm)
